# baseline (speedup 1.0000x reference)
import functools
from functools import partial

import jax
import jax.numpy as jnp
from jax.experimental import pallas as pl
from jax.experimental.pallas import tpu as pltpu


def _rup(x, m):
    return (x + m - 1) // m * m


NB = 256
EBLK = 4096
NSEG = 3


def _k1_body(x_ref, w_ref, al_ref, ar_ref, out_ref, *, Hr, F):
    f = jnp.dot(x_ref[...], w_ref[...], preferred_element_type=jnp.float32)
    out_ref[...] = jnp.zeros_like(out_ref)
    for hh in range(Hr):
        fh = f[:, hh * F:(hh + 1) * F]
        el = jnp.sum(fh * al_ref[hh:hh + 1, 0:F], axis=1, keepdims=True)
        er = jnp.sum(fh * ar_ref[hh:hh + 1, 0:F], axis=1, keepdims=True)
        out_ref[:, hh, 0:F] = fh
        out_ref[:, hh, F:F + 1] = jnp.ones_like(el)
        out_ref[:, hh, F + 1:F + 2] = el
        out_ref[:, hh, F + 2:F + 3] = er
    return


def _k2_body(src_ref, dst_ref, feat_ref, o_ref, *, F, SEG, base):
    eb = pl.program_id(0)

    @pl.when(eb == 0)
    def _():
        o_ref[...] = jnp.zeros_like(o_ref)

    def body(i, carry):
        s = src_ref[0, 0, i]
        d = dst_ref[0, 0, i]
        fs = feat_ref[s]
        fd = feat_ref[d]
        logit = fs[:, F + 1:F + 2] + fd[:, F + 2:F + 3]
        z = jnp.where(logit > 0, logit, 0.2 * logit)
        w = jnp.exp(z)
        ld = d - base
        valid = (ld >= 0) & (ld < SEG)
        w = jnp.where(valid, w, 0.0)
        ldc = jnp.clip(ld, 0, SEG - 1)
        o_ref[ldc] += w * fs
        return carry

    jax.lax.fori_loop(0, EBLK, body, 0)


def _k3_body(o_ref, b_ref, out_ref, *, Hr, F, act, DP):
    o = o_ref[...]
    out_ref[...] = jnp.zeros_like(out_ref)
    for hh in range(Hr):
        s = o[:, hh, F:F + 1]
        res = o[:, hh, 0:F] / (s + 1e-9) + b_ref[hh:hh + 1, 0:F]
        if act:
            res = jnp.where(res > 0, res, jnp.exp(jnp.minimum(res, 0.0)) - 1.0)
        out_ref[:, hh * F:(hh + 1) * F] = res


def _k4_body(v_ref, w_ref, o_ref):
    @pl.when(pl.program_id(0) == 0)
    def _():
        o_ref[...] = jnp.zeros_like(o_ref)

    o_ref[...] += jnp.dot(v_ref[...], w_ref[...],
                          preferred_element_type=jnp.float32)


def _k5_body(a_ref, b1_ref, w2_ref, b2_ref, w3_ref, o_ref):
    r1 = jnp.maximum(a_ref[...] + b1_ref[...], 0.0)
    r2 = jnp.dot(r1, w2_ref[...], preferred_element_type=jnp.float32)
    r2 = jnp.maximum(r2 + b2_ref[...], 0.0)
    o_ref[...] = jnp.dot(r2, w3_ref[...], preferred_element_type=jnp.float32)


def _gat_layer(xp, srcp, dstp, Wp, alp, arp, bp, Hr, F, act, NP, NEB, SEG):
    Din = xp.shape[1]
    DPout = Wp.shape[1]
    feat = pl.pallas_call(
        partial(_k1_body, Hr=Hr, F=F),
        grid=(NP // NB,),
        in_specs=[
            pl.BlockSpec((NB, Din), lambda k: (k, 0)),
            pl.BlockSpec((Din, DPout), lambda k: (0, 0)),
            pl.BlockSpec((Hr, 128), lambda k: (0, 0)),
            pl.BlockSpec((Hr, 128), lambda k: (0, 0)),
        ],
        out_specs=pl.BlockSpec((NB, 8, 128), lambda k: (k, 0, 0)),
        out_shape=jax.ShapeDtypeStruct((NP, 8, 128), jnp.float32),
    )(xp, Wp, alp, arp)
    o_segs = []
    for p in range(NSEG):
        o_segs.append(pl.pallas_call(
            partial(_k2_body, F=F, SEG=SEG, base=p * SEG),
            grid=(NEB,),
            in_specs=[
                pl.BlockSpec((1, 1, EBLK), lambda eb: (eb, 0, 0),
                             memory_space=pltpu.SMEM),
                pl.BlockSpec((1, 1, EBLK), lambda eb: (eb, 0, 0),
                             memory_space=pltpu.SMEM),
                pl.BlockSpec((NP, 8, 128), lambda eb: (0, 0, 0)),
            ],
            out_specs=pl.BlockSpec((SEG, 8, 128), lambda eb: (0, 0, 0)),
            out_shape=jax.ShapeDtypeStruct((SEG, 8, 128), jnp.float32),
        )(srcp, dstp, feat))
    o2 = jnp.concatenate(o_segs, axis=0)[:NP]
    xnext = pl.pallas_call(
        partial(_k3_body, Hr=Hr, F=F, act=act, DP=DPout),
        grid=(NP // NB,),
        in_specs=[
            pl.BlockSpec((NB, 8, 128), lambda k: (k, 0, 0)),
            pl.BlockSpec((Hr, 128), lambda k: (0, 0)),
        ],
        out_specs=pl.BlockSpec((NB, DPout), lambda k: (k, 0)),
        out_shape=jax.ShapeDtypeStruct((NP, DPout), jnp.float32),
    )(o2, bp)
    return xnext


def kernel(h, f, edge_index, W0, al0, ar0, b0, W1, al1, ar1, b1,
           W2, al2, ar2, b2, W3, al3, ar3, b3, Wf1, bf1, Wf2, bf2, Wf3, bf3):
    N, IN = h.shape
    E = edge_index.shape[1]
    H, HID = al0.shape
    GOUT = al3.shape[1]
    D = H * HID

    NP = _rup(N + 1, NB)
    SEG = _rup((NP + NSEG - 1) // NSEG, 8)
    NEB = (E + EBLK - 1) // EBLK
    EP = NEB * EBLK
    INP = _rup(IN, 128)
    DP = _rup(D, 128)
    GP = 128

    dummy = jnp.int32(N)
    srcp = jnp.concatenate(
        [edge_index[0], jnp.full((EP - E,), dummy, jnp.int32)]
    ).reshape(NEB, 1, EBLK)
    dstp = jnp.concatenate(
        [edge_index[1], jnp.full((EP - E,), dummy, jnp.int32)]
    ).reshape(NEB, 1, EBLK)

    def padw(W, rp, cp):
        return jnp.pad(W, ((0, rp - W.shape[0]), (0, cp - W.shape[1])))

    def pada(a, F):
        return jnp.pad(a, ((0, 0), (0, 128 - F)))

    def padb(b, Hr, F):
        return jnp.pad(b.reshape(Hr, F), ((0, 0), (0, 128 - F)))

    xp = jnp.pad(h, ((0, NP - N), (0, INP - IN)))
    x = _gat_layer(xp, srcp, dstp, padw(W0, INP, DP), pada(al0, HID),
                   pada(ar0, HID), padb(b0, H, HID), H, HID, True,
                   NP, NEB, SEG)
    x = _gat_layer(x, srcp, dstp, padw(W1, DP, DP), pada(al1, HID),
                   pada(ar1, HID), padb(b1, H, HID), H, HID, True,
                   NP, NEB, SEG)
    x = _gat_layer(x, srcp, dstp, padw(W2, DP, DP), pada(al2, HID),
                   pada(ar2, HID), padb(b2, H, HID), H, HID, True,
                   NP, NEB, SEG)
    x3 = _gat_layer(x, srcp, dstp, padw(W3, DP, GP), pada(al3, GOUT),
                    pada(ar3, GOUT), padb(b3, 1, GOUT), 1, GOUT, False,
                    NP, NEB, SEG)

    VL = N * GOUT + IN
    KB = 4096
    NKB = (VL + KB - 1) // KB
    VP = NKB * KB
    v = jnp.concatenate([x3[:N, :GOUT].reshape(-1), f,
                         jnp.zeros((VP - VL,), jnp.float32)]).reshape(1, VP)
    Wf1p = padw(Wf1, VP, 512)
    acc = pl.pallas_call(
        _k4_body,
        grid=(NKB,),
        in_specs=[
            pl.BlockSpec((1, KB), lambda k: (0, k)),
            pl.BlockSpec((KB, 512), lambda k: (k, 0)),
        ],
        out_specs=pl.BlockSpec((1, 512), lambda k: (0, 0)),
        out_shape=jax.ShapeDtypeStruct((1, 512), jnp.float32),
    )(v, Wf1p)
    res = pl.pallas_call(
        _k5_body,
        in_specs=[
            pl.BlockSpec((1, 512), lambda: (0, 0)),
            pl.BlockSpec((1, 512), lambda: (0, 0)),
            pl.BlockSpec((512, 512), lambda: (0, 0)),
            pl.BlockSpec((1, 512), lambda: (0, 0)),
            pl.BlockSpec((512, 128), lambda: (0, 0)),
        ],
        out_specs=pl.BlockSpec((1, 128), lambda: (0, 0)),
        out_shape=jax.ShapeDtypeStruct((1, 128), jnp.float32),
    )(acc, jnp.pad(bf1, (0, 12)).reshape(1, 512),
      padw(Wf2, 512, 512), jnp.pad(bf2, (0, 12)).reshape(1, 512),
      padw(Wf3, 512, 128))
    return res[0, 0:1] + bf3


# speedup vs baseline: 15.7361x; 15.7361x over previous
import functools
from functools import partial

import jax
import jax.numpy as jnp
from jax import lax
from jax.experimental import pallas as pl
from jax.experimental.pallas import tpu as pltpu
from jax.experimental.pallas import tpu_sc as plsc


def _rup(x, m):
    return (x + m - 1) // m * m


NB = 256
EB = 64


def _k1_body(x_ref, w_ref, al_ref, ar_ref, out_ref, elr_ref, *, Hr, F):
    f = jnp.dot(x_ref[...], w_ref[...], preferred_element_type=jnp.float32)
    out_ref[...] = jnp.zeros_like(out_ref)
    elr_ref[...] = jnp.zeros_like(elr_ref)
    for hh in range(Hr):
        fh = f[:, hh * F:(hh + 1) * F]
        el = jnp.sum(fh * al_ref[hh:hh + 1, 0:F], axis=1, keepdims=True)
        er = jnp.sum(fh * ar_ref[hh:hh + 1, 0:F], axis=1, keepdims=True)
        out_ref[:, hh, 0:F] = fh
        out_ref[:, hh, F:F + 1] = jnp.ones_like(el)
        out_ref[:, hh, F + 1:F + 2] = el
        out_ref[:, hh, F + 2:F + 3] = er
        el16 = lax.bitcast_convert_type(
            lax.convert_element_type(el, jnp.bfloat16), jnp.uint16)
        er16 = lax.bitcast_convert_type(
            lax.convert_element_type(er, jnp.bfloat16), jnp.uint16)
        word = lax.bitwise_or(
            lax.shift_left(er16.astype(jnp.uint32), jnp.uint32(16)),
            el16.astype(jnp.uint32))
        elr_ref[:, hh:hh + 1] = lax.bitcast_convert_type(word, jnp.int32)


def _sc_edge(feat_cm, elr_flat, srcp, dstp, Hr, NP, EP):
    NCH = Hr
    CPS = (NCH + 1) // 2
    EPT = EP // 16
    NBATCH = EPT // EB
    SEGN = 4864
    segs = []
    s0 = 0
    while s0 < NP:
        segs.append((s0, min(SEGN, NP - s0)))
        s0 += SEGN
    mesh = plsc.VectorSubcoreMesh(core_axis_name="c", subcore_axis_name="s")

    @functools.partial(
        pl.kernel, mesh=mesh,
        compiler_params=pltpu.CompilerParams(needs_layout_passes=False),
        out_type=jax.ShapeDtypeStruct((NCH, NP, 128), jnp.float32),
        scratch_types=[
            pltpu.VMEM((NP * Hr,), jnp.int32),
            pltpu.VMEM((EB,), jnp.int32),
            pltpu.VMEM((EB,), jnp.int32),
            pltpu.VMEM((EB,), jnp.int32),
            pltpu.VMEM((EB,), jnp.float32),
            pltpu.VMEM((EB, 128), jnp.float32),
            pltpu.VMEM((8, 128), jnp.float32),
            pltpu.VMEM_SHARED((SEGN + 64, 128), jnp.float32),
            pltpu.SemaphoreType.DMA,
        ])
    def k(feat_hbm, elr_hbm, src_hbm, dst_hbm, o_hbm,
          elr_v, src_v, dst_v, idx_v, w_v, rows_v, zero_v, o_sh, sem):
        cid = lax.axis_index("c")
        sid = lax.axis_index("s")
        pltpu.sync_copy(elr_hbm, elr_v)
        for i in range(8):
            for g in range(8):
                zero_v[i, g * 16:(g + 1) * 16] = jnp.zeros((16,), jnp.float32)
        for ci in range(CPS):
            c = cid * CPS + ci
            valid_chunk = c < NCH
            for segbase, seglen in segs:
                rpt = seglen // 16

                @pl.when(valid_chunk)
                def _():
                    def zbody(z, _z):
                        pltpu.sync_copy(
                            zero_v, o_sh.at[pl.ds(sid * rpt + z * 8, 8)])
                        return _z
                    lax.fori_loop(0, rpt // 8, zbody, 0)
                plsc.subcore_barrier()

                def batch_body(b, _):
                    base_e = sid * EPT + b * EB
                    pltpu.sync_copy(src_hbm.at[pl.ds(base_e, EB)], src_v)
                    pltpu.sync_copy(dst_hbm.at[pl.ds(base_e, EB)], dst_v)

                    def wbody(j2, _2):
                        s16 = src_v[pl.ds(j2 * 16, 16)]
                        d16 = dst_v[pl.ds(j2 * 16, 16)]
                        idx_v[pl.ds(j2 * 16, 16)] = s16 * NCH + c
                        ew = plsc.load_gather(elr_v, [s16 * Hr + c])
                        rw = plsc.load_gather(elr_v, [d16 * Hr + c])
                        elf = plsc.bitcast(lax.shift_left(ew, 16), jnp.float32)
                        erf = plsc.bitcast(
                            lax.bitwise_and(rw, jnp.int32(-65536)),
                            jnp.float32)
                        lg = elf + erf
                        zz = jnp.where(lg > 0, lg, 0.2 * lg)
                        w_v[pl.ds(j2 * 16, 16)] = jnp.exp(zz)
                        ld = d16 - segbase
                        ok = (ld >= 0) & (ld < seglen)
                        dst_v[pl.ds(j2 * 16, 16)] = jnp.where(
                            ok, ld, jnp.int32(SEGN))
                        return 0

                    lax.fori_loop(0, EB // 16, wbody, 0)
                    pltpu.async_copy(feat_hbm.at[idx_v], rows_v, sem).wait()

                    def sbody(j4, _2):
                        for u in range(4):
                            j = j4 * 4 + u
                            wv = plsc.load_gather(
                                w_v, [jnp.full((16,), j, jnp.int32)])
                            for g in range(8):
                                sl = pl.ds(g * 16, 16)
                                rows_v[j, sl] = rows_v[j, sl] * wv
                        return 0

                    lax.fori_loop(0, EB // 4, sbody, 0)
                    pltpu.sync_copy(rows_v, o_sh.at[dst_v], add=True)
                    return 0

                @pl.when(valid_chunk)
                def _():
                    lax.fori_loop(0, NBATCH, batch_body, 0)
                plsc.subcore_barrier()

                @pl.when(valid_chunk)
                def _():
                    pltpu.sync_copy(
                        o_sh.at[pl.ds(sid * rpt, rpt)],
                        o_hbm.at[c, pl.ds(segbase + sid * rpt, rpt)])
                plsc.subcore_barrier()

    return k(feat_cm, elr_flat, srcp, dstp)


def _k3_body(o_ref, b_ref, out_ref, *, Hr, F, act):
    out_ref[...] = jnp.zeros_like(out_ref)
    for hh in range(Hr):
        oh = o_ref[hh]
        s = oh[:, F:F + 1]
        res = oh[:, 0:F] / (s + 1e-9) + b_ref[hh:hh + 1, 0:F]
        if act:
            res = jnp.where(res > 0, res, jnp.exp(jnp.minimum(res, 0.0)) - 1.0)
        out_ref[:, hh * F:(hh + 1) * F] = res


def _k4_body(v_ref, w_ref, o_ref):
    @pl.when(pl.program_id(0) == 0)
    def _():
        o_ref[...] = jnp.zeros_like(o_ref)

    o_ref[...] += jnp.dot(v_ref[...], w_ref[...],
                          preferred_element_type=jnp.float32)


def _k5_body(a_ref, b1_ref, w2_ref, b2_ref, w3_ref, o_ref):
    r1 = jnp.maximum(a_ref[...] + b1_ref[...], 0.0)
    r2 = jnp.dot(r1, w2_ref[...], preferred_element_type=jnp.float32)
    r2 = jnp.maximum(r2 + b2_ref[...], 0.0)
    o_ref[...] = jnp.dot(r2, w3_ref[...], preferred_element_type=jnp.float32)


def _gat_layer(xp, srcp, dstp, Wp, alp, arp, bp, Hr, F, act, NP, EP):
    Din = xp.shape[1]
    DPout = Wp.shape[1]
    feat, elr = pl.pallas_call(
        partial(_k1_body, Hr=Hr, F=F),
        grid=(NP // NB,),
        in_specs=[
            pl.BlockSpec((NB, Din), lambda k: (k, 0)),
            pl.BlockSpec((Din, DPout), lambda k: (0, 0)),
            pl.BlockSpec((Hr, 128), lambda k: (0, 0)),
            pl.BlockSpec((Hr, 128), lambda k: (0, 0)),
        ],
        out_specs=[
            pl.BlockSpec((NB, 8, 128), lambda k: (k, 0, 0)),
            pl.BlockSpec((NB, 8), lambda k: (k, 0)),
        ],
        out_shape=[
            jax.ShapeDtypeStruct((NP, 8, 128), jnp.float32),
            jax.ShapeDtypeStruct((NP, 8), jnp.int32),
        ],
    )(xp, Wp, alp, arp)
    feat_cm = feat[:, :Hr, :].reshape(NP * Hr, 128)
    elr_flat = elr[:, :Hr].reshape(-1)
    o16 = _sc_edge(feat_cm, elr_flat, srcp, dstp, Hr, NP, EP)
    xnext = pl.pallas_call(
        partial(_k3_body, Hr=Hr, F=F, act=act),
        grid=(NP // NB,),
        in_specs=[
            pl.BlockSpec((Hr, NB, 128), lambda k: (0, k, 0)),
            pl.BlockSpec((Hr, 128), lambda k: (0, 0)),
        ],
        out_specs=pl.BlockSpec((NB, DPout), lambda k: (k, 0)),
        out_shape=jax.ShapeDtypeStruct((NP, DPout), jnp.float32),
    )(o16, bp)
    return xnext


def kernel(h, f, edge_index, W0, al0, ar0, b0, W1, al1, ar1, b1,
           W2, al2, ar2, b2, W3, al3, ar3, b3, Wf1, bf1, Wf2, bf2, Wf3, bf3):
    N, IN = h.shape
    E = edge_index.shape[1]
    H, HID = al0.shape
    GOUT = al3.shape[1]
    D = H * HID

    NP = _rup(N + 1, max(NB, 1024))
    EP = _rup(E, 16 * EB)
    INP = _rup(IN, 128)
    DP = _rup(D, 128)
    GP = 128

    dummy = jnp.int32(N)
    srcp = jnp.concatenate(
        [edge_index[0], jnp.full((EP - E,), dummy, jnp.int32)])
    dstp = jnp.concatenate(
        [edge_index[1], jnp.full((EP - E,), dummy, jnp.int32)])

    def padw(W, rp, cp):
        return jnp.pad(W, ((0, rp - W.shape[0]), (0, cp - W.shape[1])))

    def pada(a, F):
        return jnp.pad(a, ((0, 0), (0, 128 - F)))

    def padb(b, Hr, F):
        return jnp.pad(b.reshape(Hr, F), ((0, 0), (0, 128 - F)))

    xp = jnp.pad(h, ((0, NP - N), (0, INP - IN)))
    x = _gat_layer(xp, srcp, dstp, padw(W0, INP, DP), pada(al0, HID),
                   pada(ar0, HID), padb(b0, H, HID), H, HID, True, NP, EP)
    x = _gat_layer(x, srcp, dstp, padw(W1, DP, DP), pada(al1, HID),
                   pada(ar1, HID), padb(b1, H, HID), H, HID, True, NP, EP)
    x = _gat_layer(x, srcp, dstp, padw(W2, DP, DP), pada(al2, HID),
                   pada(ar2, HID), padb(b2, H, HID), H, HID, True, NP, EP)
    x3 = _gat_layer(x, srcp, dstp, padw(W3, DP, GP), pada(al3, GOUT),
                    pada(ar3, GOUT), padb(b3, 1, GOUT), 1, GOUT, False,
                    NP, EP)

    VL = N * GOUT + IN
    KB = 4096
    NKB = (VL + KB - 1) // KB
    VP = NKB * KB
    v = jnp.concatenate([x3[:N, :GOUT].reshape(-1), f,
                         jnp.zeros((VP - VL,), jnp.float32)]).reshape(1, VP)
    Wf1p = padw(Wf1, VP, 512)
    acc = pl.pallas_call(
        _k4_body,
        grid=(NKB,),
        in_specs=[
            pl.BlockSpec((1, KB), lambda k: (0, k)),
            pl.BlockSpec((KB, 512), lambda k: (k, 0)),
        ],
        out_specs=pl.BlockSpec((1, 512), lambda k: (0, 0)),
        out_shape=jax.ShapeDtypeStruct((1, 512), jnp.float32),
    )(v, Wf1p)
    res = pl.pallas_call(
        _k5_body,
        in_specs=[
            pl.BlockSpec((1, 512), lambda: (0, 0)),
            pl.BlockSpec((1, 512), lambda: (0, 0)),
            pl.BlockSpec((512, 512), lambda: (0, 0)),
            pl.BlockSpec((1, 512), lambda: (0, 0)),
            pl.BlockSpec((512, 128), lambda: (0, 0)),
        ],
        out_specs=pl.BlockSpec((1, 128), lambda: (0, 0)),
        out_shape=jax.ShapeDtypeStruct((1, 128), jnp.float32),
    )(acc, jnp.pad(bf1, (0, 12)).reshape(1, 512),
      padw(Wf2, 512, 512), jnp.pad(bf2, (0, 12)).reshape(1, 512),
      padw(Wf3, 512, 128))
    return res[0, 0:1] + bf3
